# baseline (speedup 1.0000x reference)
import functools

import jax
import jax.numpy as jnp
from jax import lax
from jax.experimental import pallas as pl
from jax.experimental.pallas import tpu as pltpu
from jax.experimental.pallas import tpu_sc as plsc

D = 128
H = 64
N = 10000
E = 320000
EP = E // 2
NL = 6
NSP = 3
CUTOFF = 6.0

NC = 2
NS = 16
CH = 80
CHP = CH // 2
EPT = E // NS
NCH = EPT // CH
NH = N // 2
NDM = 8
ACCR = NH + NDM
RPT = 312
ZB = 104

BN = 1000
BEP = 2000


def _init_body(x_ref, d_ref, emb_ref, h_ref, e_ref):
    i = pl.program_id(0)
    c = pl.program_id(1)

    @pl.when(c == 0)
    def _():
        xb = x_ref[...]
        acc = jnp.zeros((1, 1), jnp.float32)
        for sp in range(NSP):
            m = (xb == sp).astype(jnp.float32)
            acc = acc + m * emb_ref[sp:sp + 1, :]
        h_ref[...] = jnp.broadcast_to(acc, (BN, D))

    del i
    d = d_ref[...]
    n = (lax.broadcasted_iota(jnp.int32, (1, H), 1)
         + 1 + H * c).astype(jnp.float32)
    scale = jnp.float32((2.0 / CUTOFF) ** 0.5)
    w = jnp.float32(jnp.pi / CUTOFF)

    def bes(dcol):
        return scale * jnp.sin(n * (w * dcol)) / (dcol + 1e-8)

    e_ref[...] = jnp.concatenate([bes(d[:, 0:1]), bes(d[:, 1:2])],
                                 axis=1)[None]


def _tc_init(x, d2, emb):
    grid = (max(N // BN, EP // BEP), NC)
    return pl.pallas_call(
        _init_body,
        grid=grid,
        in_specs=[
            pl.BlockSpec((BN, 1), lambda i, c: (i % (N // BN), 0)),
            pl.BlockSpec((BEP, 2), lambda i, c: (i, 0)),
            pl.BlockSpec((NSP, D), lambda i, c: (0, 0)),
        ],
        out_specs=[
            pl.BlockSpec((BN, D), lambda i, c: (i % (N // BN), 0)),
            pl.BlockSpec((1, BEP, D), lambda i, c: (c, i, 0)),
        ],
        out_shape=[
            jax.ShapeDtypeStruct((N, D), jnp.float32),
            jax.ShapeDtypeStruct((NC, EP, D), jnp.float32),
        ],
    )(x, d2, emb)


def _tab_body(h_ref, wb_ref, wd_ref, we_ref, bb_ref, bd_ref, be_ref,
              wa_ref, ba_ref, ta_ref, te_ref, ha_ref):
    h = h_ref[...]
    hb = jnp.dot(h, wb_ref[0], preferred_element_type=jnp.float32) \
        + bb_ref[0]
    hd = jnp.dot(h, wd_ref[0], preferred_element_type=jnp.float32) \
        + bd_ref[0]
    he = jnp.dot(h, we_ref[...], preferred_element_type=jnp.float32) \
        + be_ref[...]
    ta_ref[...] = jnp.concatenate([hb, hd], axis=1)
    te_ref[...] = he
    ha_ref[...] = jnp.dot(h, wa_ref[...],
                          preferred_element_type=jnp.float32) + ba_ref[...]


def _tc_tab(h, WBi, WDi, WEi, bBi, bDi, bEi, WAi, bAi):
    grid = (N // BN, NC)
    wsplit = lambda w: jnp.stack([w[:, :H], w[:, H:]])
    bsplit = lambda b: b.reshape(NC, 1, H)
    return pl.pallas_call(
        _tab_body,
        grid=grid,
        in_specs=[
            pl.BlockSpec((BN, D), lambda i, c: (i, 0)),
            pl.BlockSpec((1, D, H), lambda i, c: (c, 0, 0)),
            pl.BlockSpec((1, D, H), lambda i, c: (c, 0, 0)),
            pl.BlockSpec((D, D), lambda i, c: (0, 0)),
            pl.BlockSpec((1, 1, H), lambda i, c: (c, 0, 0)),
            pl.BlockSpec((1, 1, H), lambda i, c: (c, 0, 0)),
            pl.BlockSpec((1, D), lambda i, c: (0, 0)),
            pl.BlockSpec((D, D), lambda i, c: (0, 0)),
            pl.BlockSpec((1, D), lambda i, c: (0, 0)),
        ],
        out_specs=[
            pl.BlockSpec((BN, D), lambda i, c: (c * (N // BN) + i, 0)),
            pl.BlockSpec((BN, D), lambda i, c: (i, 0)),
            pl.BlockSpec((BN, D), lambda i, c: (i, 0)),
        ],
        out_shape=[
            jax.ShapeDtypeStruct((NC * N, D), jnp.float32),
            jax.ShapeDtypeStruct((N, D), jnp.float32),
            jax.ShapeDtypeStruct((N, D), jnp.float32),
        ],
    )(h, wsplit(WBi), wsplit(WDi), WEi, bsplit(bBi), bsplit(bDi),
      bEi.reshape(1, D), WAi, bAi.reshape(1, D))


def _ec_body(e_ref, wa_ref, wb_ref, bc_ref, out_ref):
    p = e_ref[...]
    out = jnp.dot(p[0], wa_ref[0], preferred_element_type=jnp.float32) \
        + jnp.dot(p[1], wb_ref[0], preferred_element_type=jnp.float32) \
        + bc_ref[0]
    out_ref[...] = out[None]


def _tc_ec(e2p, WCAd, WCBd, bCp):
    grid = (EP // BEP, NC)
    return pl.pallas_call(
        _ec_body,
        grid=grid,
        in_specs=[
            pl.BlockSpec((NC, BEP, D), lambda i, c: (0, i, 0)),
            pl.BlockSpec((1, D, D), lambda i, c: (c, 0, 0)),
            pl.BlockSpec((1, D, D), lambda i, c: (c, 0, 0)),
            pl.BlockSpec((1, 1, D), lambda i, c: (c, 0, 0)),
        ],
        out_specs=pl.BlockSpec((1, BEP, D), lambda i, c: (c, i, 0)),
        out_shape=jax.ShapeDtypeStruct((NC, EP, D), jnp.float32),
    )(e2p, WCAd, WCBd, bCp)


def _upd_body(h_ref, ha_ref, nd_ref, out_ref):
    nd = nd_ref[...]
    num = jnp.concatenate([nd[0, :, 0:H], nd[1, :, 0:H]], axis=1)
    den = jnp.concatenate([nd[0, :, H:D], nd[1, :, H:D]], axis=1) + 1e-6
    v = ha_ref[...] + num / den
    sg = 1.0 / (1.0 + jnp.exp(-v))
    out_ref[...] = h_ref[...] + v * sg


def _tc_upd(h, hA, numden):
    grid = (N // BN,)
    return pl.pallas_call(
        _upd_body,
        grid=grid,
        in_specs=[
            pl.BlockSpec((BN, D), lambda i: (i, 0)),
            pl.BlockSpec((BN, D), lambda i: (i, 0)),
            pl.BlockSpec((NC, BN, D), lambda i: (0, i, 0)),
        ],
        out_specs=pl.BlockSpec((BN, D), lambda i: (i, 0)),
        out_shape=jax.ShapeDtypeStruct((N, D), jnp.float32),
    )(h, hA, numden)


def _final_body(h_ref, ha_ref, nd_ref, wh_ref, bh_ref, wo_ref, bo_ref,
                out_ref):
    nd = nd_ref[...]
    num = jnp.concatenate([nd[0, :, 0:H], nd[1, :, 0:H]], axis=1)
    den = jnp.concatenate([nd[0, :, H:D], nd[1, :, H:D]], axis=1) + 1e-6
    v = ha_ref[...] + num / den
    h = h_ref[...] + v * (1.0 / (1.0 + jnp.exp(-v)))
    t = jnp.dot(h, wh_ref[...], preferred_element_type=jnp.float32) \
        + bh_ref[...]
    t = t * (1.0 / (1.0 + jnp.exp(-t)))
    o = jnp.sum(t * wo_ref[...], axis=1, keepdims=True) + bo_ref[...]
    out_ref[...] = 1.0 / (1.0 + jnp.exp(-o))


def _tc_final(h, hA, numden, Wh, bh, Wo, bo):
    grid = (N // BN,)
    return pl.pallas_call(
        _final_body,
        grid=grid,
        in_specs=[
            pl.BlockSpec((BN, D), lambda i: (i, 0)),
            pl.BlockSpec((BN, D), lambda i: (i, 0)),
            pl.BlockSpec((NC, BN, D), lambda i: (0, i, 0)),
            pl.BlockSpec((D, D), lambda i: (0, 0)),
            pl.BlockSpec((1, D), lambda i: (0, 0)),
            pl.BlockSpec((1, D), lambda i: (0, 0)),
            pl.BlockSpec((1, 1), lambda i: (0, 0)),
        ],
        out_specs=pl.BlockSpec((BN, 1), lambda i: (i, 0)),
        out_shape=jax.ShapeDtypeStruct((N, 1), jnp.float32),
    )(h, hA, numden, Wh, bh.reshape(1, D), Wo.reshape(1, D),
      bo.reshape(1, 1))


def _sc_edge_body(e2p, ec2p, tabA, tabE, srcA2, dst2, s0v,
                  enew2p, numden,
                  sidx, didx, ebuf, ecbuf, gabuf, gebuf, msbuf,
                  zbuf, s0vm, acc, sem):
    c = lax.axis_index("c")
    s = lax.axis_index("s")

    pltpu.sync_copy(s0v, s0vm)
    lane = lax.broadcasted_iota(jnp.int32, (16,), 0)
    s0 = s0vm[...][0]

    b = jnp.minimum(jnp.maximum(s0 - s * EPT, 0), EPT)
    ka_end = (b + CH - 1) // CH
    kb_start = b // CH

    def zero_acc():
        def zcp(i, carry):
            off = pl.multiple_of(s * RPT + i * ZB, 8)
            pltpu.sync_copy(zbuf, acc.at[pl.ds(off, ZB)])
            return carry
        lax.fori_loop(0, RPT // ZB, zcp, 0)

        @pl.when(s == 0)
        def _():
            pltpu.sync_copy(zbuf.at[pl.ds(0, ACCR - RPT * NS)],
                            acc.at[pl.ds(pl.multiple_of(RPT * NS, 8),
                                         ACCR - RPT * NS)])

    def drain_acc(phase):
        soff = pl.multiple_of(s * RPT, 8)
        doff = pl.multiple_of(c * N + phase * NH + s * RPT, 8)
        pltpu.sync_copy(acc.at[pl.ds(soff, RPT)],
                        numden.at[pl.ds(doff, RPT)])

        @pl.when(s == 0)
        def _():
            toff = pl.multiple_of(RPT * NS, 8)
            pltpu.sync_copy(
                acc.at[pl.ds(toff, NH - RPT * NS)],
                numden.at[pl.ds(pl.multiple_of(
                    c * N + phase * NH + RPT * NS, 8), NH - RPT * NS)])

    def make_chunk(phase):
      def chunk(k, carry):
        ib = pl.multiple_of(c * E + s * EPT + k * CH, 8)
        pb = pl.multiple_of(c * EP + s * (EPT // 2) + k * CHP, 8)
        pltpu.sync_copy(srcA2.at[pl.ds(ib, CH)], sidx)
        pltpu.sync_copy(dst2.at[pl.ds(ib, CH)], didx)
        d1 = pltpu.async_copy(tabA.at[sidx], gabuf, sem)
        d2 = pltpu.async_copy(tabE.at[didx], gebuf, sem)
        d3 = pltpu.async_copy(e2p.at[pl.ds(pb, CHP)], ebuf, sem)
        d4 = pltpu.async_copy(ec2p.at[pl.ds(pb, CHP)], ecbuf, sem)
        d1.wait()
        d2.wait()
        d3.wait()
        d4.wait()

        def edge(j2, carry2):
            for k8 in range(8):
                je = 2 * j2 + (k8 // 4)
                kf = k8 % 4
                sl_e = pl.ds(16 * k8, 16)
                sl_b = pl.ds(16 * kf, 16)
                sl_d = pl.ds(H + 16 * kf, 16)
                sl_g = pl.ds(H * c + 16 * kf, 16)
                ehat = ecbuf[j2, sl_e] + gabuf[je, sl_d] + gebuf[je, sl_g]
                sg = 1.0 / (1.0 + jnp.exp(-ehat))
                msbuf[je, sl_b] = sg * gabuf[je, sl_b]
                msbuf[je, sl_d] = sg
                ebuf[j2, sl_e] = ebuf[j2, sl_e] + ehat * sg
            return carry2
        lax.fori_loop(0, CHP, edge, 0)

        dummy = NH + (lane & (NDM - 1))
        for t in range(CH // 16):
            v = didx[pl.ds(16 * t, 16)]
            if phase == 0:
                nv = jnp.where(v < NH, v, dummy)
            else:
                nv = jnp.where(v >= NH, v - NH, dummy)
            didx[pl.ds(16 * t, 16)] = nv

        pltpu.sync_copy(msbuf, acc.at[didx], add=True)
        pltpu.sync_copy(ebuf, enew2p.at[pl.ds(pb, CHP)])
        return carry
      return chunk

    def zrow(j, carry):
        for k in range(D // 16):
            zbuf[j, pl.ds(16 * k, 16)] = jnp.zeros((16,), jnp.float32)
        return carry
    lax.fori_loop(0, ZB, zrow, 0)

    zero_acc()
    plsc.subcore_barrier()
    lax.fori_loop(0, ka_end, make_chunk(0), 0)
    plsc.subcore_barrier()
    drain_acc(0)
    plsc.subcore_barrier()
    zero_acc()
    plsc.subcore_barrier()
    lax.fori_loop(kb_start, NCH, make_chunk(1), 0)
    plsc.subcore_barrier()
    drain_acc(1)


@functools.lru_cache(maxsize=1)
def _sc_edge_kernel():
  return pl.kernel(
    _sc_edge_body,
    out_type=[
        jax.ShapeDtypeStruct((E, D), jnp.float32),
        jax.ShapeDtypeStruct((NC * N, D), jnp.float32),
    ],
    mesh=plsc.VectorSubcoreMesh(core_axis_name="c", subcore_axis_name="s",
                                num_cores=NC, num_subcores=NS),
    scratch_types=[
        pltpu.VMEM((CH,), jnp.int32),
        pltpu.VMEM((CH,), jnp.int32),
        pltpu.VMEM((CHP, D), jnp.float32),
        pltpu.VMEM((CHP, D), jnp.float32),
        pltpu.VMEM((CH, D), jnp.float32),
        pltpu.VMEM((CH, D), jnp.float32),
        pltpu.VMEM((CH, D), jnp.float32),
        pltpu.VMEM((ZB, D), jnp.float32),
        pltpu.VMEM((16,), jnp.int32),
        pltpu.VMEM_SHARED((ACCR, D), jnp.float32),
        pltpu.SemaphoreType.DMA,
    ],
  )


def _blkdiag(a):
    z = jnp.zeros((H, H), jnp.float32)
    return jnp.concatenate([jnp.concatenate([a, z], 1),
                            jnp.concatenate([z, a], 1)], 0)


def kernel(x, edge_index, edge_attr, emb, WA, bA, WB, bB, WC, bC, WD, bD,
           WE, bE, Wh, bh, Wo, bo):
    src = edge_index[0].astype(jnp.int32)
    dst = edge_index[1].astype(jnp.int32)

    grp = (dst >= NH).astype(jnp.int32)
    c0 = jnp.cumsum(1 - grp)
    c1 = jnp.cumsum(grp)
    s0 = c0[E - 1]
    pos = jnp.where(grp == 0, c0 - 1, s0 + c1 - 1)
    perm = jnp.zeros((E,), jnp.int32).at[pos].set(
        jnp.arange(E, dtype=jnp.int32))
    src = jnp.take(src, perm)
    dst = jnp.take(dst, perm)
    attr = jnp.take(edge_attr, perm)

    srcA2 = jnp.concatenate([src, src + N])
    dst2 = jnp.concatenate([dst, dst])
    s0v = jnp.full((16,), s0, jnp.int32)

    h, e2p = _tc_init(x.reshape(N, 1).astype(jnp.int32),
                      attr.reshape(EP, 2), emb)

    for i in range(NL):
        tabA, tabE, hA = _tc_tab(h, WB[i], WD[i], WE[i], bB[i], bD[i],
                                 bE[i], WA[i], bA[i])
        WCAd = jnp.stack([_blkdiag(WC[i][0:H, 0:H]),
                          _blkdiag(WC[i][0:H, H:D])])
        WCBd = jnp.stack([_blkdiag(WC[i][H:D, 0:H]),
                          _blkdiag(WC[i][H:D, H:D])])
        bCp = jnp.concatenate([bC[i].reshape(NC, H)] * 2,
                              axis=1).reshape(NC, 1, D)
        ec2p = _tc_ec(e2p, WCAd, WCBd, bCp)

        enew, numden = _sc_edge_kernel()(e2p.reshape(E, D),
                                         ec2p.reshape(E, D),
                                         tabA, tabE, srcA2, dst2, s0v)
        e2p = enew.reshape(NC, EP, D)
        nd = numden.reshape(NC, N, D)
        if i < NL - 1:
            h = _tc_upd(h, hA, nd)
        else:
            out = _tc_final(h, hA, nd, Wh, bh, Wo, bo)
    return out
